# Initial kernel scaffold; baseline (speedup 1.0000x reference)
import jax
import jax.numpy as jnp
from jax.experimental import pallas as pl


def kernel(x, W, b):
    raise NotImplementedError("write your pallas kernel here")


# speedup vs baseline: 2.6300x; 2.6300x over previous
import functools

import jax
import jax.numpy as jnp
from jax.experimental import pallas as pl
from jax.experimental.pallas import tpu as pltpu


def _router_body(x_ref, w_ref, b_ref, idx_ref, val_ref, aux_ref,
                 imp_acc, load_acc, *, nsteps, total_rows, n_expert):
    i = pl.program_id(0)

    logits = jax.lax.dot_general(
        x_ref[...], w_ref[...], (((1,), (1,)), ((), ())),
        preferred_element_type=jnp.float32) + b_ref[...]

    m1 = jnp.max(logits, axis=-1, keepdims=True)
    iota = jax.lax.broadcasted_iota(jnp.int32, logits.shape, 1)
    im1 = jnp.min(jnp.where(logits == m1, iota, n_expert),
                  axis=-1, keepdims=True)
    first_mask = iota == im1

    p_un = jnp.exp(logits - m1)
    s = jnp.sum(p_un, axis=-1, keepdims=True)
    p2_un = jnp.max(jnp.where(first_mask, -1.0, p_un), axis=-1, keepdims=True)
    im2 = jnp.min(jnp.where((p_un == p2_un) & ~first_mask, iota, n_expert),
                  axis=-1, keepdims=True)

    inv_s = 1.0 / s
    idx_ref[...] = jnp.concatenate([im1, im2], axis=1)
    val_ref[...] = jnp.concatenate([inv_s, p2_un * inv_s], axis=1)

    @pl.when(i == 0)
    def _init():
        imp_acc[...] = jnp.zeros_like(imp_acc)
        load_acc[...] = jnp.zeros_like(load_acc)

    imp_acc[...] += jnp.sum(p_un * inv_s, axis=0, keepdims=True)
    load_acc[...] += jnp.sum(jnp.where(first_mask, 1.0, 0.0),
                             axis=0, keepdims=True)

    @pl.when(i == nsteps - 1)
    def _fin():
        imp = imp_acc[...] * (1.0 / total_rows)
        load = load_acc[...] * (1.0 / total_rows)
        aux_ref[...] = (n_expert * jnp.sum(imp * load)).reshape(1, 1)


@jax.jit
def kernel(x, W, b):
    S, D = x.shape
    E = W.shape[0]
    R = 1024
    nsteps = S // R
    b2 = b.reshape(1, E)

    body = functools.partial(_router_body, nsteps=nsteps,
                             total_rows=S, n_expert=E)
    idx, vals, aux = pl.pallas_call(
        body,
        grid=(nsteps,),
        in_specs=[
            pl.BlockSpec((R, D), lambda i: (i, 0)),
            pl.BlockSpec((E, D), lambda i: (0, 0)),
            pl.BlockSpec((1, E), lambda i: (0, 0)),
        ],
        out_specs=[
            pl.BlockSpec((R, 2), lambda i: (i, 0)),
            pl.BlockSpec((R, 2), lambda i: (i, 0)),
            pl.BlockSpec((1, 1), lambda i: (0, 0)),
        ],
        out_shape=[
            jax.ShapeDtypeStruct((S, 2), jnp.int32),
            jax.ShapeDtypeStruct((S, 2), jnp.float32),
            jax.ShapeDtypeStruct((1, 1), jnp.float32),
        ],
        scratch_shapes=[
            pltpu.VMEM((1, E), jnp.float32),
            pltpu.VMEM((1, E), jnp.float32),
        ],
        compiler_params=pltpu.CompilerParams(
            dimension_semantics=("arbitrary",)),
    )(x, W, b2)
    return idx, vals, aux.reshape(())


# speedup vs baseline: 4.7368x; 1.8011x over previous
import functools

import jax
import jax.numpy as jnp
from jax.experimental import pallas as pl
from jax.experimental.pallas import tpu as pltpu


def _router_body(x_ref, w_ref, b_ref, idx_ref, val_ref, aux_ref,
                 imp_acc, load_acc, *, nsteps, total_rows, n_expert):
    i = pl.program_id(0)

    logits = jax.lax.dot_general(
        w_ref[...], x_ref[...], (((1,), (1,)), ((), ())),
        preferred_element_type=jnp.float32) + b_ref[...]

    m1 = jnp.max(logits, axis=0, keepdims=True)
    iota = jax.lax.broadcasted_iota(jnp.int32, logits.shape, 0)
    first_mask = logits == m1
    im1 = jnp.min(jnp.where(first_mask, iota, n_expert),
                  axis=0, keepdims=True)
    only_first = iota == im1

    p_un = jnp.exp(logits - m1)
    s = jnp.sum(p_un, axis=0, keepdims=True)
    p2_un = jnp.max(jnp.where(only_first, -1.0, p_un), axis=0, keepdims=True)
    im2 = jnp.min(jnp.where((p_un == p2_un) & ~only_first, iota, n_expert),
                  axis=0, keepdims=True)

    inv_s = 1.0 / s
    idx_ref[...] = jnp.concatenate([im1, im2], axis=0)
    val_ref[...] = jnp.concatenate([inv_s, p2_un * inv_s], axis=0)

    @pl.when(i == 0)
    def _init():
        imp_acc[...] = jnp.zeros_like(imp_acc)
        load_acc[...] = jnp.zeros_like(load_acc)

    imp_acc[...] += p_un * inv_s
    load_acc[...] += jnp.where(only_first, 1.0, 0.0)

    @pl.when(i == nsteps - 1)
    def _fin():
        inv_n = 1.0 / total_rows
        imp = jnp.sum(imp_acc[...], axis=1) * inv_n
        load = jnp.sum(load_acc[...], axis=1) * inv_n
        aux_ref[...] = (n_expert * jnp.sum(imp * load)).reshape(1, 1)


@jax.jit
def kernel(x, W, b):
    S, D = x.shape
    E = W.shape[0]
    R = 1024
    nsteps = S // R
    b2 = b.reshape(E, 1)

    body = functools.partial(_router_body, nsteps=nsteps,
                             total_rows=S, n_expert=E)
    idx, vals, aux = pl.pallas_call(
        body,
        grid=(nsteps,),
        in_specs=[
            pl.BlockSpec((R, D), lambda i: (i, 0)),
            pl.BlockSpec((E, D), lambda i: (0, 0)),
            pl.BlockSpec((E, 1), lambda i: (0, 0)),
        ],
        out_specs=[
            pl.BlockSpec((2, R), lambda i: (0, i)),
            pl.BlockSpec((2, R), lambda i: (0, i)),
            pl.BlockSpec((1, 1), lambda i: (0, 0)),
        ],
        out_shape=[
            jax.ShapeDtypeStruct((2, S), jnp.int32),
            jax.ShapeDtypeStruct((2, S), jnp.float32),
            jax.ShapeDtypeStruct((1, 1), jnp.float32),
        ],
        scratch_shapes=[
            pltpu.VMEM((E, R), jnp.float32),
            pltpu.VMEM((E, R), jnp.float32),
        ],
        compiler_params=pltpu.CompilerParams(
            dimension_semantics=("arbitrary",)),
    )(x, W, b2)
    return idx.T, vals.T, aux.reshape(())


# speedup vs baseline: 5.7991x; 1.2243x over previous
import functools

import jax
import jax.numpy as jnp
from jax.experimental import pallas as pl
from jax.experimental.pallas import tpu as pltpu


def _router_body(x_ref, w_ref, b_ref, idx_ref, val_ref, aux_ref,
                 imp_acc, load_acc, *, nsteps, total_rows, n_expert):
    i = pl.program_id(0)

    logits = jax.lax.dot_general(
        w_ref[...], x_ref[...], (((1,), (1,)), ((), ())),
        preferred_element_type=jnp.float32) + b_ref[...]

    m1 = jnp.max(logits, axis=0, keepdims=True)
    iota = jax.lax.broadcasted_iota(jnp.int32, logits.shape, 0)
    first_mask = logits == m1
    im1 = jnp.min(jnp.where(first_mask, iota, n_expert),
                  axis=0, keepdims=True)
    only_first = iota == im1

    p_un = jnp.exp(logits - m1)
    s = jnp.sum(p_un, axis=0, keepdims=True)
    p2_un = jnp.max(jnp.where(only_first, -1.0, p_un), axis=0, keepdims=True)
    im2 = jnp.min(jnp.where((p_un == p2_un) & ~only_first, iota, n_expert),
                  axis=0, keepdims=True)

    inv_s = 1.0 / s
    idx_ref[...] = jnp.concatenate([im1, im2], axis=0)
    val_ref[...] = jnp.concatenate([inv_s, p2_un * inv_s], axis=0)

    @pl.when(i == 0)
    def _init():
        imp_acc[...] = jnp.zeros_like(imp_acc)
        load_acc[...] = jnp.zeros_like(load_acc)

    imp_acc[...] += p_un * inv_s
    load_acc[...] += jnp.where(only_first, 1.0, 0.0)

    @pl.when(i == nsteps - 1)
    def _fin():
        inv_n = 1.0 / total_rows
        imp = jnp.sum(imp_acc[...], axis=1) * inv_n
        load = jnp.sum(load_acc[...], axis=1) * inv_n
        aux_ref[...] = (n_expert * jnp.sum(imp * load)).reshape(1, 1)


@jax.jit
def kernel(x, W, b):
    S, D = x.shape
    E = W.shape[0]
    R = 2048
    nsteps = S // R
    b2 = b.reshape(E, 1)

    body = functools.partial(_router_body, nsteps=nsteps,
                             total_rows=S, n_expert=E)
    idx, vals, aux = pl.pallas_call(
        body,
        grid=(nsteps,),
        in_specs=[
            pl.BlockSpec((R, D), lambda i: (i, 0)),
            pl.BlockSpec((E, D), lambda i: (0, 0)),
            pl.BlockSpec((E, 1), lambda i: (0, 0)),
        ],
        out_specs=[
            pl.BlockSpec((2, R), lambda i: (0, i)),
            pl.BlockSpec((2, R), lambda i: (0, i)),
            pl.BlockSpec((1, 1), lambda i: (0, 0)),
        ],
        out_shape=[
            jax.ShapeDtypeStruct((2, S), jnp.int32),
            jax.ShapeDtypeStruct((2, S), jnp.float32),
            jax.ShapeDtypeStruct((1, 1), jnp.float32),
        ],
        scratch_shapes=[
            pltpu.VMEM((E, R), jnp.float32),
            pltpu.VMEM((E, R), jnp.float32),
        ],
        compiler_params=pltpu.CompilerParams(
            dimension_semantics=("arbitrary",)),
    )(x, W, b2)
    return idx.T, vals.T, aux.reshape(())


# speedup vs baseline: 6.3146x; 1.0889x over previous
import functools

import jax
import jax.numpy as jnp
from jax.experimental import pallas as pl
from jax.experimental.pallas import tpu as pltpu


def _router_body(x_ref, w_ref, b_ref, idx_ref, val_ref, aux_ref,
                 imp_acc, load_acc, *, nsteps, total_rows, n_expert):
    i = pl.program_id(0)

    logits = jax.lax.dot_general(
        w_ref[...], x_ref[...], (((1,), (1,)), ((), ())),
        preferred_element_type=jnp.float32) + b_ref[...]

    m1 = jnp.max(logits, axis=0, keepdims=True)
    iota = jax.lax.broadcasted_iota(jnp.int32, logits.shape, 0)
    first_mask = logits == m1
    im1 = jnp.min(jnp.where(first_mask, iota, n_expert),
                  axis=0, keepdims=True)
    only_first = iota == im1

    p_un = jnp.exp(logits - m1)
    s = jnp.sum(p_un, axis=0, keepdims=True)
    p2_un = jnp.max(jnp.where(only_first, -1.0, p_un), axis=0, keepdims=True)
    im2 = jnp.min(jnp.where((p_un == p2_un) & ~only_first, iota, n_expert),
                  axis=0, keepdims=True)

    inv_s = 1.0 / s
    idx_ref[...] = jnp.concatenate([im1, im2], axis=0)
    val_ref[...] = jnp.concatenate([inv_s, p2_un * inv_s], axis=0)

    @pl.when(i == 0)
    def _init():
        imp_acc[...] = jnp.zeros_like(imp_acc)
        load_acc[...] = jnp.zeros_like(load_acc)

    imp_acc[...] += p_un * inv_s
    load_acc[...] += jnp.where(only_first, 1.0, 0.0)

    @pl.when(i == nsteps - 1)
    def _fin():
        inv_n = 1.0 / total_rows
        imp = jnp.sum(imp_acc[...], axis=1) * inv_n
        load = jnp.sum(load_acc[...], axis=1) * inv_n
        aux_ref[...] = (n_expert * jnp.sum(imp * load)).reshape(1, 1)


@jax.jit
def kernel(x, W, b):
    S, D = x.shape
    E = W.shape[0]
    R = 4096
    nsteps = S // R
    b2 = b.reshape(E, 1)

    body = functools.partial(_router_body, nsteps=nsteps,
                             total_rows=S, n_expert=E)
    idx, vals, aux = pl.pallas_call(
        body,
        grid=(nsteps,),
        in_specs=[
            pl.BlockSpec((R, D), lambda i: (i, 0)),
            pl.BlockSpec((E, D), lambda i: (0, 0)),
            pl.BlockSpec((E, 1), lambda i: (0, 0)),
        ],
        out_specs=[
            pl.BlockSpec((2, R), lambda i: (0, i)),
            pl.BlockSpec((2, R), lambda i: (0, i)),
            pl.BlockSpec((1, 1), lambda i: (0, 0)),
        ],
        out_shape=[
            jax.ShapeDtypeStruct((2, S), jnp.int32),
            jax.ShapeDtypeStruct((2, S), jnp.float32),
            jax.ShapeDtypeStruct((1, 1), jnp.float32),
        ],
        scratch_shapes=[
            pltpu.VMEM((E, R), jnp.float32),
            pltpu.VMEM((E, R), jnp.float32),
        ],
        compiler_params=pltpu.CompilerParams(
            dimension_semantics=("arbitrary",)),
    )(x, W, b2)
    return idx.T, vals.T, aux.reshape(())
